# Initial kernel scaffold; baseline (speedup 1.0000x reference)
import jax
import jax.numpy as jnp
from jax.experimental import pallas as pl


def kernel(hidden, beliefs, W_q, W_out, W_util, log_temperature, Wg1, bg1, Wg2, bg2, conv_w):
    raise NotImplementedError("write your pallas kernel here")


# speedup vs baseline: 1.5574x; 1.5574x over previous
import functools

import jax
import jax.numpy as jnp
from jax.experimental import pallas as pl
from jax.experimental.pallas import tpu as pltpu

_EPS = 1e-8
_TOPK = 128
_NEG = float("-inf")


def _select_body(n, nb, bn, bel_ref, rq_ref, idx_ref, s_ref):
    i = pl.program_id(0)

    @pl.when(i < nb)
    def _():
        b = bel_ref[...]
        n2 = jnp.sum(b * b, axis=1, keepdims=True)
        angles = b / jnp.maximum(jnp.sqrt(n2), _EPS)
        s = jax.lax.dot_general(rq_ref[...], angles,
                                (((1,), (1,)), ((), ())),
                                preferred_element_type=jnp.float32)
        col = i * bn + jax.lax.broadcasted_iota(jnp.int32, (1, bn), 1)
        s = jnp.where(col < n, s, _NEG)
        s_ref[pl.ds(i, 1), :] = s

    @pl.when(i == nb)
    def _():
        rows = jax.lax.broadcasted_iota(jnp.int32, (nb, bn), 0)
        cols = jax.lax.broadcasted_iota(jnp.int32, (nb, bn), 1)
        flat = rows * bn + cols
        lane = jax.lax.broadcasted_iota(jnp.int32, (1, _TOPK), 1)
        big = jnp.int32(2**31 - 1)

        def body(k, acc):
            s = s_ref[...]
            m = jnp.max(s)
            idx = jnp.min(jnp.where(s == m, flat, big))
            acc = jnp.where(lane == k, idx, acc)
            s_ref[...] = jnp.where(flat == idx, _NEG, s)
            return acc

        acc = jax.lax.fori_loop(
            0, _TOPK, body, jnp.zeros((1, _TOPK), jnp.int32))
        idx_ref[...] = acc


def _select_kernel(beliefs, rq):
    n, d = beliefs.shape
    bn = 1024
    nb = (n + bn - 1) // bn
    return pl.pallas_call(
        functools.partial(_select_body, n, nb, bn),
        grid=(nb + 1,),
        in_specs=[
            pl.BlockSpec((bn, d), lambda i: (jnp.minimum(i, nb - 1), 0)),
            pl.BlockSpec((1, d), lambda i: (0, 0)),
        ],
        out_specs=pl.BlockSpec((1, _TOPK), lambda i: (0, 0)),
        out_shape=jax.ShapeDtypeStruct((1, _TOPK), jnp.int32),
        scratch_shapes=[pltpu.VMEM((nb, bn), jnp.float32)],
        compiler_params=pltpu.CompilerParams(
            dimension_semantics=("arbitrary",)),
    )(beliefs, rq)


def _attn_body(nh, d, blocks_per_batch,
               hid_ref, vals_ref, wq_ref, wg1_ref, bg1_ref, wg2_ref,
               convw_ref, temps_ref,
               attn_ref, retr_ref, rr_ref, gate_ref, hist_ref):
    i = pl.program_id(0)
    dims = (((1,), (1,)), ((), ()))
    f32 = jnp.float32
    bm = rr_ref.shape[0]

    hb = hid_ref[...]

    g1 = jax.lax.dot_general(hb, wg1_ref[...], dims,
                             preferred_element_type=f32)
    g1 = jnp.maximum(g1 + bg1_ref[...], 0.0)
    gate_raw = jnp.sum(g1 * wg2_ref[...], axis=1, keepdims=True)
    gate_raw = gate_raw + temps_ref[nh]
    gr = jnp.sqrt(jnp.maximum(jnp.abs(gate_raw), 1e-6)) * jnp.sign(gate_raw)
    gate = jax.nn.sigmoid(gr)
    gate_ref[...] = gate

    q = jax.lax.dot_general(hb, wq_ref[...], dims,
                            preferred_element_type=f32)

    vals = vals_ref[...]
    n2 = jnp.sum(vals * vals, axis=1, keepdims=True)
    keys = vals / jnp.maximum(jnp.sqrt(n2), _EPS)

    r_parts = []
    for hh in range(nh):
        qh = q[:, hh * d:(hh + 1) * d]
        s = jax.lax.dot_general(qh, keys, dims,
                                preferred_element_type=f32)
        s = s * temps_ref[hh]
        s = s - jnp.max(s, axis=1, keepdims=True)
        p = jnp.exp(s)
        attn_h = p / jnp.sum(p, axis=1, keepdims=True)
        attn_ref[:, hh * _TOPK:(hh + 1) * _TOPK] = attn_h
        r_h = jax.lax.dot_general(attn_h, vals,
                                  (((1,), (0,)), ((), ())),
                                  preferred_element_type=f32)
        retr_ref[:, hh * d:(hh + 1) * d] = r_h
        r_parts.append(r_h)
    r = jnp.concatenate(r_parts, axis=1)

    first = (i % blocks_per_batch) == 0
    prev3 = jnp.where(first, 0.0, hist_ref[5:8, :])
    xp = jnp.concatenate([prev3, r], axis=0)
    conv = (xp[0:bm, :] * convw_ref[0:1, :]
            + xp[1:bm + 1, :] * convw_ref[1:2, :]
            + xp[2:bm + 2, :] * convw_ref[2:3, :]
            + r * convw_ref[3:4, :])
    hist_ref[...] = r[bm - 8:bm, :]
    rr_ref[...] = r + conv * jax.nn.sigmoid(conv)


def _attn_kernel(hid2, vals, wq, wg1, bg1r, wg2, convwT, temps,
                 nh, d, bm, blocks_per_batch):
    m, h = hid2.shape
    r = wq.shape[0]
    hq = wg1.shape[0]
    return pl.pallas_call(
        functools.partial(_attn_body, nh, d, blocks_per_batch),
        grid=(m // bm,),
        in_specs=[
            pl.BlockSpec((bm, h), lambda i: (i, 0)),
            pl.BlockSpec((_TOPK, d), lambda i: (0, 0)),
            pl.BlockSpec((r, h), lambda i: (0, 0)),
            pl.BlockSpec((hq, h), lambda i: (0, 0)),
            pl.BlockSpec((1, hq), lambda i: (0, 0)),
            pl.BlockSpec((1, hq), lambda i: (0, 0)),
            pl.BlockSpec((4, r), lambda i: (0, 0)),
            pl.BlockSpec(memory_space=pltpu.SMEM),
        ],
        out_specs=[
            pl.BlockSpec((bm, nh * _TOPK), lambda i: (i, 0)),
            pl.BlockSpec((bm, nh * d), lambda i: (i, 0)),
            pl.BlockSpec((bm, r), lambda i: (i, 0)),
            pl.BlockSpec((bm, 1), lambda i: (i, 0)),
        ],
        out_shape=[
            jax.ShapeDtypeStruct((m, nh * _TOPK), jnp.float32),
            jax.ShapeDtypeStruct((m, nh * d), jnp.float32),
            jax.ShapeDtypeStruct((m, r), jnp.float32),
            jax.ShapeDtypeStruct((m, 1), jnp.float32),
        ],
        scratch_shapes=[pltpu.VMEM((8, r), jnp.float32)],
        compiler_params=pltpu.CompilerParams(
            dimension_semantics=("arbitrary",)),
    )(hid2, vals, wq, wg1, bg1r, wg2, convwT, temps)


def _proj_body(rr_ref, gate_ref, wout_ref, wutil_ref, out_ref, util_ref):
    dims = (((1,), (1,)), ((), ()))
    f32 = jnp.float32
    rr = rr_ref[...]
    gate = gate_ref[...]
    out_ref[...] = jax.lax.dot_general(
        rr, wout_ref[...], dims, preferred_element_type=f32) * gate
    util_ref[...] = jax.lax.dot_general(
        rr, wutil_ref[...], dims, preferred_element_type=f32) * gate


def _proj_kernel(rr, gate, wout, wutil, bm):
    m, r = rr.shape
    h = wout.shape[0]
    return pl.pallas_call(
        _proj_body,
        grid=(m // bm,),
        in_specs=[
            pl.BlockSpec((bm, r), lambda i: (i, 0)),
            pl.BlockSpec((bm, 1), lambda i: (i, 0)),
            pl.BlockSpec((h, r), lambda i: (0, 0)),
            pl.BlockSpec((h, r), lambda i: (0, 0)),
        ],
        out_specs=[
            pl.BlockSpec((bm, h), lambda i: (i, 0)),
            pl.BlockSpec((bm, h), lambda i: (i, 0)),
        ],
        out_shape=[
            jax.ShapeDtypeStruct((m, h), jnp.float32),
            jax.ShapeDtypeStruct((m, h), jnp.float32),
        ],
        compiler_params=pltpu.CompilerParams(
            dimension_semantics=("arbitrary",)),
    )(rr, gate, wout, wutil)


def kernel(hidden, beliefs, W_q, W_out, W_util, log_temperature,
           Wg1, bg1, Wg2, bg2, conv_w):
    B, T, H = hidden.shape
    nh = log_temperature.shape[0]
    N, D = beliefs.shape
    bf16 = jnp.bfloat16

    hid2 = hidden.reshape(B * T, H)
    bg1r = bg1.reshape(1, -1)
    convwT = conv_w.T

    temperature = jnp.clip(jnp.exp(log_temperature), 0.1, None)
    scal = jnp.concatenate(
        [temperature / (D ** 0.5), bg2.reshape(1)]).astype(jnp.float32)

    bm = 256 if T % 256 == 0 else T
    blocks_per_batch = T // bm

    mean_query = hid2.mean(axis=0)
    qf = jnp.einsum("h,rh->r", mean_query.astype(bf16), W_q.astype(bf16),
                    preferred_element_type=jnp.float32)
    rq = qf.reshape(nh, D).mean(axis=0).reshape(1, D)
    idx = _select_kernel(beliefs, rq)
    vals = jnp.take(beliefs, idx[0], axis=0)

    attnf, retrf, rr, gate = _attn_kernel(
        hid2, vals, W_q, Wg1, bg1r, Wg2, convwT, scal,
        nh, D, bm, blocks_per_batch)
    out2, util2 = _proj_kernel(rr, gate, W_out, W_util, bm)

    return (out2.reshape(B, T, H),
            util2.reshape(B, T, H),
            attnf.reshape(B, T, nh, _TOPK),
            retrf.reshape(B, T, nh, D))


# speedup vs baseline: 1.5825x; 1.0161x over previous
import functools

import jax
import jax.numpy as jnp
from jax.experimental import pallas as pl
from jax.experimental.pallas import tpu as pltpu

_EPS = 1e-8
_TOPK = 128
_NEG = float("-inf")


def _select_body(n, nb, bn, bel_ref, rq_ref, idx_ref, s_ref):
    i = pl.program_id(0)

    @pl.when(i < nb)
    def _():
        b = bel_ref[...]
        n2 = jnp.sum(b * b, axis=1, keepdims=True)
        angles = b / jnp.maximum(jnp.sqrt(n2), _EPS)
        s = jax.lax.dot_general(rq_ref[...], angles,
                                (((1,), (1,)), ((), ())),
                                preferred_element_type=jnp.float32)
        col = i * bn + jax.lax.broadcasted_iota(jnp.int32, (1, bn), 1)
        s = jnp.where(col < n, s, _NEG)
        s_ref[pl.ds(i, 1), :] = s

    @pl.when(i == nb)
    def _():
        rows = jax.lax.broadcasted_iota(jnp.int32, (nb, bn), 0)
        cols = jax.lax.broadcasted_iota(jnp.int32, (nb, bn), 1)
        flat = rows * bn + cols
        lane = jax.lax.broadcasted_iota(jnp.int32, (1, _TOPK), 1)
        big = jnp.int32(2**31 - 1)

        def body(k, acc):
            s = s_ref[...]
            m = jnp.max(s)
            idx = jnp.min(jnp.where(s == m, flat, big))
            acc = jnp.where(lane == k, idx, acc)
            s_ref[...] = jnp.where(flat == idx, _NEG, s)
            return acc

        acc = jax.lax.fori_loop(
            0, _TOPK, body, jnp.zeros((1, _TOPK), jnp.int32))
        idx_ref[...] = acc


def _select_kernel(beliefs, rq):
    n, d = beliefs.shape
    bn = 1024
    nb = (n + bn - 1) // bn
    return pl.pallas_call(
        functools.partial(_select_body, n, nb, bn),
        grid=(nb + 1,),
        in_specs=[
            pl.BlockSpec((bn, d), lambda i: (jnp.minimum(i, nb - 1), 0)),
            pl.BlockSpec((1, d), lambda i: (0, 0)),
        ],
        out_specs=pl.BlockSpec((1, _TOPK), lambda i: (0, 0)),
        out_shape=jax.ShapeDtypeStruct((1, _TOPK), jnp.int32),
        scratch_shapes=[pltpu.VMEM((nb, bn), jnp.float32)],
        compiler_params=pltpu.CompilerParams(
            dimension_semantics=("arbitrary",)),
    )(beliefs, rq)


def _attn_body(nh, d, blocks_per_batch,
               hid_ref, vals_ref, wq_ref, wg1_ref, bg1_ref, wg2_ref,
               convw_ref, temps_ref,
               attn_ref, retr_ref, rr_ref, gate_ref, hist_ref):
    i = pl.program_id(0)
    dims = (((1,), (1,)), ((), ()))
    f32 = jnp.float32
    bm = rr_ref.shape[0]

    hb = hid_ref[...]

    g1 = jax.lax.dot_general(hb, wg1_ref[...], dims,
                             preferred_element_type=f32)
    g1 = jnp.maximum(g1 + bg1_ref[...], 0.0)
    gate_raw = jnp.sum(g1 * wg2_ref[...], axis=1, keepdims=True)
    gate_raw = gate_raw + temps_ref[nh]
    gr = jnp.sqrt(jnp.maximum(jnp.abs(gate_raw), 1e-6)) * jnp.sign(gate_raw)
    gate = jax.nn.sigmoid(gr)
    gate_ref[...] = gate

    q = jax.lax.dot_general(hb, wq_ref[...], dims,
                            preferred_element_type=f32)

    vals = vals_ref[...]
    n2 = jnp.sum(vals * vals, axis=1, keepdims=True)
    keys = vals / jnp.maximum(jnp.sqrt(n2), _EPS)

    r_parts = []
    for hh in range(nh):
        qh = q[:, hh * d:(hh + 1) * d]
        s = jax.lax.dot_general(qh, keys, dims,
                                preferred_element_type=f32)
        s = s * temps_ref[hh]
        s = s - jnp.max(s, axis=1, keepdims=True)
        p = jnp.exp(s)
        attn_h = p / jnp.sum(p, axis=1, keepdims=True)
        attn_ref[:, hh, :] = attn_h
        r_h = jax.lax.dot_general(attn_h, vals,
                                  (((1,), (0,)), ((), ())),
                                  preferred_element_type=f32)
        retr_ref[:, hh, :] = r_h
        r_parts.append(r_h)
    r = jnp.concatenate(r_parts, axis=1)

    first = (i % blocks_per_batch) == 0
    prev3 = jnp.where(first, 0.0, hist_ref[5:8, :])
    xp = jnp.concatenate([prev3, r], axis=0)
    conv = (xp[0:bm, :] * convw_ref[0:1, :]
            + xp[1:bm + 1, :] * convw_ref[1:2, :]
            + xp[2:bm + 2, :] * convw_ref[2:3, :]
            + r * convw_ref[3:4, :])
    hist_ref[...] = r[bm - 8:bm, :]
    rr_ref[...] = r + conv * jax.nn.sigmoid(conv)


def _attn_kernel(hid2, vals, wq, wg1, bg1r, wg2, convwT, temps,
                 nh, d, bm, blocks_per_batch):
    m, h = hid2.shape
    r = wq.shape[0]
    hq = wg1.shape[0]
    return pl.pallas_call(
        functools.partial(_attn_body, nh, d, blocks_per_batch),
        grid=(m // bm,),
        in_specs=[
            pl.BlockSpec((bm, h), lambda i: (i, 0)),
            pl.BlockSpec((_TOPK, d), lambda i: (0, 0)),
            pl.BlockSpec((r, h), lambda i: (0, 0)),
            pl.BlockSpec((hq, h), lambda i: (0, 0)),
            pl.BlockSpec((1, hq), lambda i: (0, 0)),
            pl.BlockSpec((1, hq), lambda i: (0, 0)),
            pl.BlockSpec((4, r), lambda i: (0, 0)),
            pl.BlockSpec(memory_space=pltpu.SMEM),
        ],
        out_specs=[
            pl.BlockSpec((bm, nh, _TOPK), lambda i: (i, 0, 0)),
            pl.BlockSpec((bm, nh, d), lambda i: (i, 0, 0)),
            pl.BlockSpec((bm, r), lambda i: (i, 0)),
            pl.BlockSpec((bm, 1), lambda i: (i, 0)),
        ],
        out_shape=[
            jax.ShapeDtypeStruct((m, nh, _TOPK), jnp.float32),
            jax.ShapeDtypeStruct((m, nh, d), jnp.float32),
            jax.ShapeDtypeStruct((m, r), jnp.float32),
            jax.ShapeDtypeStruct((m, 1), jnp.float32),
        ],
        scratch_shapes=[pltpu.VMEM((8, r), jnp.float32)],
        compiler_params=pltpu.CompilerParams(
            dimension_semantics=("arbitrary",)),
    )(hid2, vals, wq, wg1, bg1r, wg2, convwT, temps)


def _proj_body(rr_ref, gate_ref, wout_ref, wutil_ref, out_ref, util_ref):
    dims = (((1,), (1,)), ((), ()))
    f32 = jnp.float32
    rr = rr_ref[...]
    gate = gate_ref[...]
    out_ref[...] = jax.lax.dot_general(
        rr, wout_ref[...], dims, preferred_element_type=f32) * gate
    util_ref[...] = jax.lax.dot_general(
        rr, wutil_ref[...], dims, preferred_element_type=f32) * gate


def _proj_kernel(rr, gate, wout, wutil, bm):
    m, r = rr.shape
    h = wout.shape[0]
    return pl.pallas_call(
        _proj_body,
        grid=(m // bm,),
        in_specs=[
            pl.BlockSpec((bm, r), lambda i: (i, 0)),
            pl.BlockSpec((bm, 1), lambda i: (i, 0)),
            pl.BlockSpec((h, r), lambda i: (0, 0)),
            pl.BlockSpec((h, r), lambda i: (0, 0)),
        ],
        out_specs=[
            pl.BlockSpec((bm, h), lambda i: (i, 0)),
            pl.BlockSpec((bm, h), lambda i: (i, 0)),
        ],
        out_shape=[
            jax.ShapeDtypeStruct((m, h), jnp.float32),
            jax.ShapeDtypeStruct((m, h), jnp.float32),
        ],
        compiler_params=pltpu.CompilerParams(
            dimension_semantics=("arbitrary",)),
    )(rr, gate, wout, wutil)


def kernel(hidden, beliefs, W_q, W_out, W_util, log_temperature,
           Wg1, bg1, Wg2, bg2, conv_w):
    B, T, H = hidden.shape
    nh = log_temperature.shape[0]
    N, D = beliefs.shape
    bf16 = jnp.bfloat16

    hid2 = hidden.reshape(B * T, H)
    bg1r = bg1.reshape(1, -1)
    convwT = conv_w.T

    temperature = jnp.clip(jnp.exp(log_temperature), 0.1, None)
    scal = jnp.concatenate(
        [temperature / (D ** 0.5), bg2.reshape(1)]).astype(jnp.float32)

    bm = 256 if T % 256 == 0 else T
    blocks_per_batch = T // bm

    mean_query = hid2.mean(axis=0)
    qf = jnp.einsum("h,rh->r", mean_query.astype(bf16), W_q.astype(bf16),
                    preferred_element_type=jnp.float32)
    rq = qf.reshape(nh, D).mean(axis=0).reshape(1, D)
    idx = _select_kernel(beliefs, rq)
    vals = jnp.take(beliefs, idx[0], axis=0)

    attnf, retrf, rr, gate = _attn_kernel(
        hid2, vals, W_q, Wg1, bg1r, Wg2, convwT, scal,
        nh, D, bm, blocks_per_batch)
    out2, util2 = _proj_kernel(rr, gate, W_out, W_util, bm)

    return (out2.reshape(B, T, H),
            util2.reshape(B, T, H),
            attnf.reshape(B, T, nh, _TOPK),
            retrf.reshape(B, T, nh, D))
